# Initial kernel scaffold; baseline (speedup 1.0000x reference)
import jax
import jax.numpy as jnp
from jax.experimental import pallas as pl


def kernel(xyz, local_coordinates, neighbors, data_idxes, W_sa1, b_sa1, g_sa1, be_sa1, W_sa2, b_sa2, g_sa2, be_sa2, W_m1, b_m1, g_m1, be_m1, W_m2, b_m2, g_m2, be_m2, W_fc1, b_fc1, g_bn1, be_bn1, W_fc3, b_fc3):
    raise NotImplementedError("write your pallas kernel here")


# speedup vs baseline: 12.1835x; 12.1835x over previous
import functools

import jax
import jax.numpy as jnp
import numpy as np
from jax.experimental import pallas as pl
from jax.experimental.pallas import tpu as pltpu

_POINT_NUM = [2048, 512, 512, 128]
_B = 64
_N1 = 512
_K = 32
_N2 = 128
_EPS = 1e-5


def _k1_body(lc_ref, g_ref, be_ref, b1_ref, w1t_ref, h1_ref, stats_ref,
             s_ref, sq_ref):
    i = pl.program_id(0)
    x = lc_ref[0]
    w1t = w1t_ref[...]
    h = jax.lax.dot_general(w1t, x, (((1,), (0,)), ((), ())),
                            preferred_element_type=jnp.float32,
                            precision=jax.lax.Precision.HIGHEST)
    m = h[:, 0:_N1]
    for j in range(1, _K):
        m = jnp.maximum(m, h[:, j * _N1:(j + 1) * _N1])
    m = m + b1_ref[...]
    h1_ref[0] = m

    ps = jnp.sum(m, axis=1, keepdims=True)
    psq = jnp.sum(m * m, axis=1, keepdims=True)

    @pl.when(i == 0)
    def _():
        s_ref[...] = jnp.zeros_like(s_ref)
        sq_ref[...] = jnp.zeros_like(sq_ref)

    s_ref[...] += ps
    sq_ref[...] += psq

    @pl.when(i == pl.num_programs(0) - 1)
    def _():
        cnt = float(_B * _N1)
        mean = s_ref[...] / cnt
        var = sq_ref[...] / cnt - mean * mean
        scale = g_ref[...] * jax.lax.rsqrt(var + _EPS)
        shift = be_ref[...] - mean * scale
        stats_ref[...] = jnp.concatenate(
            [scale, shift, jnp.zeros((64, 6), jnp.float32)], axis=1)


def _k3_body(h1_ref, stats_ref, lc3_ref, nb_ref, di1_ref, di3_ref, xyz_ref,
             w2at_ref, w2bt_ref, b2_ref, h2_ref, xyz2_ref):
    scale = stats_ref[:, 0:1]
    shift = stats_ref[:, 1:2]
    h1n = jnp.maximum(h1_ref[0] * scale + shift, 0.0)
    h1nb = h1n.astype(jnp.bfloat16)

    nbv = nb_ref[0]
    rows = jax.lax.broadcasted_iota(jnp.int32, (_N1, _K * _N2), 0)
    oh = jnp.where(rows == nbv, 1.0, 0.0).astype(jnp.bfloat16)
    gp = jax.lax.dot_general(h1nb, oh, (((1,), (0,)), ((), ())),
                             preferred_element_type=jnp.float32)

    lc3 = lc3_ref[0]
    h2 = (jax.lax.dot_general(w2at_ref[...], lc3.astype(jnp.bfloat16),
                              (((1,), (0,)), ((), ())),
                              preferred_element_type=jnp.float32)
          + jax.lax.dot_general(w2bt_ref[...], gp.astype(jnp.bfloat16),
                                (((1,), (0,)), ((), ())),
                                preferred_element_type=jnp.float32))
    m = h2[:, 0:_N2]
    for j in range(1, _K):
        m = jnp.maximum(m, h2[:, j * _N2:(j + 1) * _N2])
    h2_ref[0] = m + b2_ref[...]

    di3v = di3_ref[0]
    di1v = di1_ref[0].astype(jnp.float32)
    iota_i = jax.lax.broadcasted_iota(jnp.int32, (_N1, _N2), 0)
    oh_a = jnp.where(iota_i == di3v, 1.0, 0.0)
    di13 = jax.lax.dot_general(di1v, oh_a, (((1,), (0,)), ((), ())),
                               preferred_element_type=jnp.float32,
                               precision=jax.lax.Precision.HIGHEST)
    oh_b = jnp.where(iota_i.astype(jnp.float32) == di13, 1.0, 0.0)
    xyz2_ref[0] = jax.lax.dot_general(xyz_ref[0], oh_b,
                                      (((1,), (0,)), ((), ())),
                                      preferred_element_type=jnp.float32,
                                      precision=jax.lax.Precision.HIGHEST)


def _k4_body(h2_ref, xyz2_ref, g2_ref, be2_ref,
             wm1_ref, bm1_ref, gm1_ref, bem1_ref,
             wm2_ref, bm2_ref, gm2_ref, bem2_ref,
             wf1_ref, bf1_ref, gb1_ref, beb1_ref,
             wf3_ref, bf3_ref, out_ref):
    cols = float(_B * _N2)

    def bn_cols(x, g, be):
        mean = jnp.mean(x, axis=1, keepdims=True)
        var = jnp.mean(x * x, axis=1, keepdims=True) - mean * mean
        scale = g * jax.lax.rsqrt(var + _EPS)
        return x * scale + (be - mean * scale)

    h2n = jnp.maximum(bn_cols(h2_ref[...], g2_ref[...], be2_ref[...]), 0.0)
    hm = jnp.concatenate([xyz2_ref[...], h2n], axis=0)
    m1 = jax.lax.dot_general(wm1_ref[...], hm, (((1,), (0,)), ((), ())),
                             preferred_element_type=jnp.float32,
                             precision=jax.lax.Precision.HIGHEST) + bm1_ref[...]
    m1 = jnp.maximum(bn_cols(m1, gm1_ref[...], bem1_ref[...]), 0.0)
    m2 = jax.lax.dot_general(wm2_ref[...], m1, (((1,), (0,)), ((), ())),
                             preferred_element_type=jnp.float32,
                             precision=jax.lax.Precision.HIGHEST) + bm2_ref[...]
    m2 = jnp.maximum(bn_cols(m2, gm2_ref[...], bem2_ref[...]), 0.0)

    parts = [jnp.max(m2[:, b * _N2:(b + 1) * _N2], axis=1, keepdims=True)
             for b in range(_B)]
    g = jnp.concatenate(parts, axis=1)

    x = jax.lax.dot_general(wf1_ref[...], g, (((1,), (0,)), ((), ())),
                            preferred_element_type=jnp.float32,
                            precision=jax.lax.Precision.HIGHEST) + bf1_ref[...]
    x = jnp.maximum(bn_cols(x, gb1_ref[...], beb1_ref[...]), 0.0)
    x = jax.lax.dot_general(wf3_ref[...], x, (((1,), (0,)), ((), ())),
                            preferred_element_type=jnp.float32,
                            precision=jax.lax.Precision.HIGHEST) + bf3_ref[...]
    mx = jnp.max(x, axis=0, keepdims=True)
    xs = x - mx
    lse = jnp.log(jnp.sum(jnp.exp(xs), axis=0, keepdims=True))
    out_ref[...] = xs - lse


def kernel(xyz, local_coordinates, neighbors, data_idxes,
           W_sa1, b_sa1, g_sa1, be_sa1,
           W_sa2, b_sa2, g_sa2, be_sa2,
           W_m1, b_m1, g_m1, be_m1,
           W_m2, b_m2, g_m2, be_m2,
           W_fc1, b_fc1, g_bn1, be_bn1,
           W_fc3, b_fc3):
    offs = np.cumsum([0] + _POINT_NUM)
    lc1 = local_coordinates[:, offs[1]:offs[2]]
    lc3 = local_coordinates[:, offs[3]:offs[4]]
    nb3 = neighbors[:, offs[3]:offs[4]].astype(jnp.int32)
    di1 = data_idxes[:, offs[1]:offs[2]].astype(jnp.int32)
    di3 = data_idxes[:, offs[3]:offs[4]].astype(jnp.int32)

    lc1t = lc1.transpose(0, 3, 2, 1).reshape(_B, 3, _K * _N1)
    lc3t = lc3.transpose(0, 3, 2, 1).reshape(_B, 3, _K * _N2)
    nb3t = nb3.transpose(0, 2, 1).reshape(_B, 1, _K * _N2)
    di1r = di1.reshape(_B, 1, _N1)
    di3r = di3.reshape(_B, 1, _N2)
    xyzt = xyz[:, :_N1].transpose(0, 2, 1)

    w1t = W_sa1.T
    b1c = b_sa1.reshape(64, 1)
    g1c = g_sa1.reshape(64, 1)
    be1c = be_sa1.reshape(64, 1)

    h1t, stats = pl.pallas_call(
        _k1_body,
        grid=(_B,),
        in_specs=[
            pl.BlockSpec((1, 3, _K * _N1), lambda i: (i, 0, 0)),
            pl.BlockSpec((64, 1), lambda i: (0, 0)),
            pl.BlockSpec((64, 1), lambda i: (0, 0)),
            pl.BlockSpec((64, 1), lambda i: (0, 0)),
            pl.BlockSpec((64, 3), lambda i: (0, 0)),
        ],
        out_specs=[
            pl.BlockSpec((1, 64, _N1), lambda i: (i, 0, 0)),
            pl.BlockSpec((64, 8), lambda i: (0, 0)),
        ],
        out_shape=[
            jax.ShapeDtypeStruct((_B, 64, _N1), jnp.float32),
            jax.ShapeDtypeStruct((64, 8), jnp.float32),
        ],
        scratch_shapes=[
            pltpu.VMEM((64, 1), jnp.float32),
            pltpu.VMEM((64, 1), jnp.float32),
        ],
    )(lc1t, g1c, be1c, b1c, w1t)

    w2at = W_sa2[:3].T
    w2bt = W_sa2[3:].T
    b2c = b_sa2.reshape(256, 1)

    h2t, xyz2t = pl.pallas_call(
        _k3_body,
        grid=(_B,),
        in_specs=[
            pl.BlockSpec((1, 64, _N1), lambda i: (i, 0, 0)),
            pl.BlockSpec((64, 8), lambda i: (0, 0)),
            pl.BlockSpec((1, 3, _K * _N2), lambda i: (i, 0, 0)),
            pl.BlockSpec((1, 1, _K * _N2), lambda i: (i, 0, 0)),
            pl.BlockSpec((1, 1, _N1), lambda i: (i, 0, 0)),
            pl.BlockSpec((1, 1, _N2), lambda i: (i, 0, 0)),
            pl.BlockSpec((1, 3, _N1), lambda i: (i, 0, 0)),
            pl.BlockSpec((256, 3), lambda i: (0, 0)),
            pl.BlockSpec((256, 64), lambda i: (0, 0)),
            pl.BlockSpec((256, 1), lambda i: (0, 0)),
        ],
        out_specs=[
            pl.BlockSpec((1, 256, _N2), lambda i: (i, 0, 0)),
            pl.BlockSpec((1, 3, _N2), lambda i: (i, 0, 0)),
        ],
        out_shape=[
            jax.ShapeDtypeStruct((_B, 256, _N2), jnp.float32),
            jax.ShapeDtypeStruct((_B, 3, _N2), jnp.float32),
        ],
    )(h1t, stats, lc3t, nb3t, di1r, di3r, xyzt,
      w2at, w2bt, b2c)

    h2cols = h2t.transpose(1, 0, 2).reshape(256, _B * _N2)
    xyz2cols = xyz2t.transpose(1, 0, 2).reshape(3, _B * _N2)

    def c(v, n):
        return v.reshape(n, 1)

    full = lambda shape: pl.BlockSpec(shape, lambda: tuple(0 for _ in shape))
    outt = pl.pallas_call(
        _k4_body,
        in_specs=[
            full((256, _B * _N2)), full((3, _B * _N2)),
            full((256, 1)), full((256, 1)),
            full((256, 259)), full((256, 1)), full((256, 1)), full((256, 1)),
            full((256, 256)), full((256, 1)), full((256, 1)), full((256, 1)),
            full((128, 256)), full((128, 1)), full((128, 1)), full((128, 1)),
            full((40, 128)), full((40, 1)),
        ],
        out_specs=full((40, _B)),
        out_shape=jax.ShapeDtypeStruct((40, _B), jnp.float32),
    )(h2cols, xyz2cols, c(g_sa2, 256), c(be_sa2, 256),
      W_m1.T, c(b_m1, 256), c(g_m1, 256), c(be_m1, 256),
      W_m2.T, c(b_m2, 256), c(g_m2, 256), c(be_m2, 256),
      W_fc1.T, c(b_fc1, 128), c(g_bn1, 128), c(be_bn1, 128),
      W_fc3.T, c(b_fc3, 40))

    return outt.T
